# Initial kernel scaffold; baseline (speedup 1.0000x reference)
import jax
import jax.numpy as jnp
from jax.experimental import pallas as pl


def kernel(x, edge_index, batch, W1, b1, p1, W2, b2, p2, W3, b3, p3, lW1, lb1, lW2, lb2):
    raise NotImplementedError("write your pallas kernel here")


# speedup vs baseline: 1.3768x; 1.3768x over previous
import jax
import jax.numpy as jnp
from jax.experimental import pallas as pl

NG = 64
RATIO = 0.6


def _gcn_conv(h, edge_index, W, b, node_mask, edge_mask):
    N = h.shape[0]
    src, dst = edge_index[0], edge_index[1]
    w = edge_mask.astype(h.dtype)
    nm = node_mask.astype(h.dtype)
    deg = jnp.zeros((N,), h.dtype).at[dst].add(w) + nm
    dinv = jnp.where(deg > 0, deg ** -0.5, 0.0)
    hW = h @ W
    msg = hW[src] * (dinv[src] * dinv[dst] * w)[:, None]
    out = jnp.zeros((N, W.shape[1]), h.dtype).at[dst].add(msg)
    out = out + hW * (dinv * dinv * nm)[:, None]
    return out + b


def _topk_keep(score, batch, node_mask):
    n = score.shape[0]
    bigb = jnp.where(node_mask, batch, NG)
    order = jnp.lexsort((-score, bigb))
    counts_all = jnp.zeros((NG + 1,), jnp.int32).at[bigb].add(1)
    starts = jnp.cumsum(counts_all) - counts_all
    bg = bigb[order]
    ranks = jnp.arange(n) - starts[bg]
    k = jnp.ceil(RATIO * counts_all[:NG].astype(jnp.float32)).astype(jnp.int32)
    k_ext = jnp.concatenate([k, jnp.zeros((1,), jnp.int32)])
    keep_sorted = ranks < k_ext[bg]
    keep = jnp.zeros((n,), bool).at[order].set(keep_sorted)
    return keep


def _readout(x, batch, node_mask):
    bb = jnp.where(node_mask, batch, NG)
    mx = jax.ops.segment_max(x, bb, num_segments=NG + 1)[:NG]
    sm = jax.ops.segment_sum(x, bb, num_segments=NG + 1)[:NG]
    cnt = jax.ops.segment_sum(jnp.ones((x.shape[0],), x.dtype), bb, num_segments=NG + 1)[:NG]
    mean = sm / jnp.maximum(cnt, 1.0)[:, None]
    return jnp.concatenate([mx, mean], axis=1)


def kernel(x, edge_index, batch, W1, b1, p1, W2, b2, p2, W3, b3, p3, lW1, lb1, lW2, lb2):
    N = x.shape[0]
    node_mask = jnp.ones((N,), bool)
    edge_mask = jnp.ones((edge_index.shape[1],), bool)
    src, dst = edge_index[0], edge_index[1]
    h = x
    outs = []
    for (W, b, p) in ((W1, b1, p1), (W2, b2, p2), (W3, b3, p3)):
        h = jax.nn.relu(_gcn_conv(h, edge_index, W, b, node_mask, edge_mask))
        score = jnp.tanh((h @ p) / jnp.linalg.norm(p))
        keep = _topk_keep(score, batch, node_mask)
        h = h * score[:, None]
        node_mask = keep
        edge_mask = edge_mask & keep[src] & keep[dst]
        outs.append(_readout(h, batch, node_mask))
    z = outs[0] + outs[1] + outs[2]
    z = jax.nn.relu(z @ lW1.T + lb1)
    return jax.nn.sigmoid(z @ lW2.T + lb2)


# speedup vs baseline: 17.6759x; 12.8381x over previous
import functools

import jax
import jax.numpy as jnp
from jax import lax
from jax.experimental import pallas as pl
from jax.experimental.pallas import tpu as pltpu
from jax.experimental.pallas import tpu_sc as plsc

N = 100000
E = 1600000
NG = 64
RATIO = 0.6
D = 32
NC = 2
NS = 16
NW = NC * NS
HALF = N // NC
CHUNK = 2048
SUB = 128
NSUB = CHUNK // SUB
ZP = 2048
E_PAD = E + 2 * CHUNK

EW_A = E // NW
NCH_A = (EW_A + CHUNK - 1) // CHUNK
EW_B = E // NS
NCH_B = (EW_B + CHUNK - 1) // CHUNK

_mesh = plsc.VectorSubcoreMesh(core_axis_name="c", subcore_axis_name="s")
_sc_params = pltpu.CompilerParams(needs_layout_passes=False)
_sc_params_b = pltpu.CompilerParams(needs_layout_passes=False,
                                    use_tc_tiling_on_sc=False)


@functools.partial(
    pl.kernel,
    out_type=(
        jax.ShapeDtypeStruct((E_PAD,), jnp.float32),
        jax.ShapeDtypeStruct((NC, N), jnp.float32),
    ),
    mesh=_mesh,
    scratch_types=[
        pltpu.VMEM((N,), jnp.float32),
        pltpu.VMEM((CHUNK,), jnp.int32),
        pltpu.VMEM((CHUNK,), jnp.int32),
        pltpu.VMEM((CHUNK,), jnp.float32),
        pltpu.VMEM((CHUNK,), jnp.float32),
        pltpu.VMEM((NSUB, SUB), jnp.int32),
        pltpu.VMEM_SHARED((N,), jnp.float32),
        pltpu.SemaphoreType.DMA,
    ],
    compiler_params=_sc_params,
)
def _edge_mask_deg(src_h, dst_h, em_h, keep_h, zn_h, em_out, deg_out,
                   keep_v, srcb, dstb, emb, emnb, didx, deg_s, sem):
    c = lax.axis_index("c")
    s = lax.axis_index("s")
    wid = c * NS + s

    @pl.when(s == 0)
    def _():
        pltpu.sync_copy(zn_h, deg_s)
    pltpu.sync_copy(keep_h, keep_v)
    plsc.subcore_barrier()

    base_w = wid * EW_A
    iota = lax.iota(jnp.int32, 16)

    def chunk_body(i, _):
        base = base_w + i * CHUNK
        pltpu.sync_copy(src_h.at[pl.ds(base, CHUNK)], srcb)
        pltpu.sync_copy(dst_h.at[pl.ds(base, CHUNK)], dstb)
        pltpu.sync_copy(em_h.at[pl.ds(base, CHUNK)], emb)

        def vec_body(j, _):
            off = j * 16
            s16 = srcb[pl.ds(off, 16)]
            d16 = dstb[pl.ds(off, 16)]
            e16 = emb[pl.ds(off, 16)]
            eid = base + off + iota
            ks = plsc.load_gather(keep_v, [s16])
            kd = plsc.load_gather(keep_v, [d16])
            emn = jnp.where(eid < E, e16 * ks * kd, 0.0)
            emnb[pl.ds(off, 16)] = emn
            r = j // 8
            cs = (j % 8) * 16
            didx[r, pl.ds(cs, 16)] = d16
            return _

        lax.fori_loop(0, CHUNK // 16, vec_body, None)
        pltpu.sync_copy(emnb, em_out.at[pl.ds(base, CHUNK)])

        def sc_body(j, _):
            pltpu.sync_copy(emnb.at[pl.ds(j * SUB, SUB)],
                            deg_s.at[didx.at[j]], add=True)
            return _

        lax.fori_loop(0, NSUB, sc_body, None)
        return _

    lax.fori_loop(0, NCH_A, chunk_body, None)
    plsc.subcore_barrier()

    @pl.when(s == 0)
    def _():
        pltpu.sync_copy(deg_s, deg_out.at[c])


@functools.partial(
    pl.kernel,
    out_type=jax.ShapeDtypeStruct((N, D), jnp.float32),
    mesh=_mesh,
    scratch_types=[
        pltpu.VMEM((CHUNK,), jnp.int32),
        pltpu.VMEM((CHUNK,), jnp.int32),
        pltpu.VMEM((CHUNK,), jnp.float32),
        pltpu.VMEM((NSUB, SUB), jnp.int32),
        pltpu.VMEM((NSUB, SUB), jnp.int32),
        pltpu.VMEM((SUB, D), jnp.float32),
        pltpu.VMEM_SHARED((HALF, D), jnp.float32),
        pltpu.SemaphoreType.DMA,
        pltpu.SemaphoreType.DMA,
    ],
    compiler_params=_sc_params_b,
)
def _messages(src_h, dst_h, em_h, g_h, zh_h, acc_out,
              srcb, dstb, emb, gidx, didx, rows, acc_s, gsem, ssem):
    c = lax.axis_index("c")
    s = lax.axis_index("s")

    @pl.when(s == 0)
    def _():
        pltpu.sync_copy(zh_h, acc_s)
    plsc.subcore_barrier()

    lo = c * HALF
    iota = lax.iota(jnp.int32, 16)

    def chunk_body(i, _):
        base = s * EW_B + i * CHUNK
        pltpu.sync_copy(src_h.at[pl.ds(base, CHUNK)], srcb)
        pltpu.sync_copy(dst_h.at[pl.ds(base, CHUNK)], dstb)
        pltpu.sync_copy(em_h.at[pl.ds(base, CHUNK)], emb)

        def vec_body(j, _):
            off = j * 16
            s16 = srcb[pl.ds(off, 16)]
            d16 = dstb[pl.ds(off, 16)]
            e16 = emb[pl.ds(off, 16)]
            eid = base + off + iota
            owned = (d16 >= lo) & (d16 < lo + HALF)
            m = (e16 > 0.5) & owned & (eid < E)
            zidx = N + (eid & (ZP - 1))
            srcm = jnp.where(m, s16, zidx)
            dloc = jnp.clip(d16 - lo, 0, HALF - 1)
            r = j // 8
            cs = (j % 8) * 16
            gidx[r, pl.ds(cs, 16)] = srcm
            didx[r, pl.ds(cs, 16)] = dloc
            return _

        lax.fori_loop(0, CHUNK // 16, vec_body, None)

        def sc_body(j, _):
            pltpu.async_copy(g_h.at[gidx.at[j]], rows, gsem).wait()
            pltpu.sync_copy(rows, acc_s.at[didx.at[j]], add=True)
            return _

        lax.fori_loop(0, NSUB, sc_body, None)
        return _

    lax.fori_loop(0, NCH_B, chunk_body, None)
    plsc.subcore_barrier()

    @pl.when(s == 0)
    def _():
        pltpu.sync_copy(acc_s, acc_out.at[pl.ds(lo, HALF)])


BR = 4000
GRID = N // BR


def _pre_body(h_ref, w_ref, d0_ref, d1_ref, k_ref, g_ref, dinv_ref):
    deg = d0_ref[...] + d1_ref[...] + k_ref[...]
    dinv = jnp.where(deg > 0, lax.rsqrt(deg), 0.0)
    hw = jnp.dot(h_ref[...], w_ref[...], preferred_element_type=jnp.float32)
    g_ref[...] = hw * dinv
    dinv_ref[...] = dinv


def _tc_pre(h, W, d0, d1, keep):
    din = h.shape[1]
    row = lambda i: (i, 0)
    full = lambda i: (0, 0)
    return pl.pallas_call(
        _pre_body,
        grid=(GRID,),
        in_specs=[
            pl.BlockSpec((BR, din), row),
            pl.BlockSpec((din, D), full),
            pl.BlockSpec((BR, 1), row),
            pl.BlockSpec((BR, 1), row),
            pl.BlockSpec((BR, 1), row),
        ],
        out_specs=(pl.BlockSpec((BR, D), row), pl.BlockSpec((BR, 1), row)),
        out_shape=(
            jax.ShapeDtypeStruct((N, D), jnp.float32),
            jax.ShapeDtypeStruct((N, 1), jnp.float32),
        ),
    )(h, W, d0, d1, keep)


def _post_body(acc_ref, g_ref, dinv_ref, k_ref, b_ref, p_ref,
               h_ref, sc_ref):
    dinv = dinv_ref[...]
    pre = (dinv * acc_ref[...] + (dinv * dinv * k_ref[...]) * g_ref[...]
           + b_ref[...])
    hr = jnp.maximum(pre, 0.0)
    sc = jnp.tanh(jnp.dot(hr, p_ref[...], preferred_element_type=jnp.float32))
    h_ref[...] = hr * sc
    sc_ref[...] = sc


def _tc_post(acc, g, dinv, keep, b, p_unit):
    row = lambda i: (i, 0)
    full = lambda i: (0, 0)
    return pl.pallas_call(
        _post_body,
        grid=(GRID,),
        in_specs=[
            pl.BlockSpec((BR, D), row),
            pl.BlockSpec((BR, D), row),
            pl.BlockSpec((BR, 1), row),
            pl.BlockSpec((BR, 1), row),
            pl.BlockSpec((1, D), full),
            pl.BlockSpec((D, 1), full),
        ],
        out_specs=(pl.BlockSpec((BR, D), row), pl.BlockSpec((BR, 1), row)),
        out_shape=(
            jax.ShapeDtypeStruct((N, D), jnp.float32),
            jax.ShapeDtypeStruct((N, 1), jnp.float32),
        ),
    )(acc, g, dinv, keep, b, p_unit)


def _topk_keep(score, batch, node_mask):
    n = score.shape[0]
    bigb = jnp.where(node_mask, batch, NG)
    order = jnp.lexsort((-score, bigb))
    counts_all = jnp.zeros((NG + 1,), jnp.int32).at[bigb].add(1)
    starts = jnp.cumsum(counts_all) - counts_all
    bg = bigb[order]
    ranks = jnp.arange(n) - starts[bg]
    k = jnp.ceil(RATIO * counts_all[:NG].astype(jnp.float32)).astype(jnp.int32)
    k_ext = jnp.concatenate([k, jnp.zeros((1,), jnp.int32)])
    keep_sorted = ranks < k_ext[bg]
    return jnp.zeros((n,), bool).at[order].set(keep_sorted)


def _readout(x, batch, node_mask):
    bb = jnp.where(node_mask, batch, NG)
    mx = jax.ops.segment_max(x, bb, num_segments=NG + 1)[:NG]
    sm = jax.ops.segment_sum(x, bb, num_segments=NG + 1)[:NG]
    cnt = jax.ops.segment_sum(jnp.ones((x.shape[0],), x.dtype), bb,
                              num_segments=NG + 1)[:NG]
    mean = sm / jnp.maximum(cnt, 1.0)[:, None]
    return jnp.concatenate([mx, mean], axis=1)


def kernel(x, edge_index, batch, W1, b1, p1, W2, b2, p2, W3, b3, p3,
           lW1, lb1, lW2, lb2):
    pad = jnp.zeros((E_PAD - E,), jnp.int32)
    src_p = jnp.concatenate([edge_index[0], pad])
    dst_p = jnp.concatenate([edge_index[1], pad])
    em = jnp.ones((E_PAD,), jnp.float32)
    keep_b = jnp.ones((N,), bool)
    zn = jnp.zeros((N,), jnp.float32)
    zh = jnp.zeros((HALF, D), jnp.float32)

    h = x
    outs = []
    for (W, b, p) in ((W1, b1, p1), (W2, b2, p2), (W3, b3, p3)):
        keep_f = keep_b.astype(jnp.float32)
        em, deg_parts = _edge_mask_deg(src_p, dst_p, em, keep_f, zn)
        g, dinv = _tc_pre(h, W, deg_parts[0][:, None], deg_parts[1][:, None],
                          keep_f[:, None])
        g_pad = jnp.concatenate([g, jnp.zeros((ZP, D), jnp.float32)], axis=0)
        acc = _messages(src_p, dst_p, em, g_pad, zh)
        p_unit = (p / jnp.linalg.norm(p))[:, None]
        h, score = _tc_post(acc, g, dinv, keep_f[:, None], b[None, :], p_unit)
        score = score[:, 0]
        keep_b = _topk_keep(score, batch, keep_b)
        outs.append(_readout(h, batch, keep_b))

    z = outs[0] + outs[1] + outs[2]
    z = jax.nn.relu(z @ lW1.T + lb1)
    return jax.nn.sigmoid(z @ lW2.T + lb2)


# speedup vs baseline: 17.8714x; 1.0111x over previous
import functools

import jax
import jax.numpy as jnp
from jax import lax
from jax.experimental import pallas as pl
from jax.experimental.pallas import tpu as pltpu
from jax.experimental.pallas import tpu_sc as plsc

N = 100000
E = 1600000
NG = 64
RATIO = 0.6
D = 32
NC = 2
NS = 16
NW = NC * NS
HALF = N // NC
CHUNK = 2048
SUB = 128
NSUB = CHUNK // SUB
ZP = 2048
E_PAD = E + 2 * CHUNK

EW_A = E // NW
NCH_A = (EW_A + CHUNK - 1) // CHUNK
EW_B = E // NS
NCH_B = (EW_B + CHUNK - 1) // CHUNK

_mesh = plsc.VectorSubcoreMesh(core_axis_name="c", subcore_axis_name="s")
_sc_params = pltpu.CompilerParams(needs_layout_passes=False)
_sc_params_b = pltpu.CompilerParams(needs_layout_passes=False,
                                    use_tc_tiling_on_sc=False)


@functools.partial(
    pl.kernel,
    out_type=(
        jax.ShapeDtypeStruct((E_PAD,), jnp.float32),
        jax.ShapeDtypeStruct((NC, N), jnp.float32),
    ),
    mesh=_mesh,
    scratch_types=[
        pltpu.VMEM((N,), jnp.float32),
        pltpu.VMEM((CHUNK,), jnp.int32),
        pltpu.VMEM((CHUNK,), jnp.int32),
        pltpu.VMEM((CHUNK,), jnp.float32),
        pltpu.VMEM((CHUNK,), jnp.float32),
        pltpu.VMEM((NSUB, SUB), jnp.int32),
        pltpu.VMEM_SHARED((N,), jnp.float32),
        pltpu.SemaphoreType.DMA,
    ],
    compiler_params=_sc_params,
)
def _edge_mask_deg(src_h, dst_h, em_h, keep_h, zn_h, em_out, deg_out,
                   keep_v, srcb, dstb, emb, emnb, didx, deg_s, sem):
    c = lax.axis_index("c")
    s = lax.axis_index("s")
    wid = c * NS + s

    @pl.when(s == 0)
    def _():
        pltpu.sync_copy(zn_h, deg_s)
    pltpu.sync_copy(keep_h, keep_v)
    plsc.subcore_barrier()

    base_w = wid * EW_A
    iota = lax.iota(jnp.int32, 16)

    def chunk_body(i, _):
        base = base_w + i * CHUNK
        pltpu.sync_copy(src_h.at[pl.ds(base, CHUNK)], srcb)
        pltpu.sync_copy(dst_h.at[pl.ds(base, CHUNK)], dstb)
        pltpu.sync_copy(em_h.at[pl.ds(base, CHUNK)], emb)

        def vec_body(j, _):
            off = j * 16
            s16 = srcb[pl.ds(off, 16)]
            d16 = dstb[pl.ds(off, 16)]
            e16 = emb[pl.ds(off, 16)]
            eid = base + off + iota
            ks = plsc.load_gather(keep_v, [s16])
            kd = plsc.load_gather(keep_v, [d16])
            emn = jnp.where(eid < E, e16 * ks * kd, 0.0)
            emnb[pl.ds(off, 16)] = emn
            r = j // 8
            cs = (j % 8) * 16
            didx[r, pl.ds(cs, 16)] = d16
            return _

        lax.fori_loop(0, CHUNK // 16, vec_body, None)
        pltpu.sync_copy(emnb, em_out.at[pl.ds(base, CHUNK)])

        sds = [pltpu.async_copy(emnb.at[pl.ds(j * SUB, SUB)],
                                deg_s.at[didx.at[j]], sem, add=True)
               for j in range(NSUB)]
        for sd in sds:
            sd.wait()
        return _

    lax.fori_loop(0, NCH_A, chunk_body, None)
    plsc.subcore_barrier()

    @pl.when(s == 0)
    def _():
        pltpu.sync_copy(deg_s, deg_out.at[c])


@functools.partial(
    pl.kernel,
    out_type=jax.ShapeDtypeStruct((N, D), jnp.float32),
    mesh=_mesh,
    scratch_types=[
        pltpu.VMEM((CHUNK,), jnp.int32),
        pltpu.VMEM((CHUNK,), jnp.int32),
        pltpu.VMEM((CHUNK,), jnp.float32),
        pltpu.VMEM((NSUB, SUB), jnp.int32),
        pltpu.VMEM((NSUB, SUB), jnp.int32),
        pltpu.VMEM((2, SUB, D), jnp.float32),
        pltpu.VMEM_SHARED((HALF, D), jnp.float32),
        pltpu.SemaphoreType.DMA,
        pltpu.SemaphoreType.DMA,
    ],
    compiler_params=_sc_params_b,
)
def _messages(src_h, dst_h, em_h, g_h, zh_h, acc_out,
              srcb, dstb, emb, gidx, didx, rows, acc_s, gsem, ssem):
    c = lax.axis_index("c")
    s = lax.axis_index("s")

    @pl.when(s == 0)
    def _():
        pltpu.sync_copy(zh_h, acc_s)
    plsc.subcore_barrier()

    lo = c * HALF
    iota = lax.iota(jnp.int32, 16)

    def chunk_body(i, _):
        base = s * EW_B + i * CHUNK
        pltpu.sync_copy(src_h.at[pl.ds(base, CHUNK)], srcb)
        pltpu.sync_copy(dst_h.at[pl.ds(base, CHUNK)], dstb)
        pltpu.sync_copy(em_h.at[pl.ds(base, CHUNK)], emb)

        def vec_body(j, _):
            off = j * 16
            s16 = srcb[pl.ds(off, 16)]
            d16 = dstb[pl.ds(off, 16)]
            e16 = emb[pl.ds(off, 16)]
            eid = base + off + iota
            owned = (d16 >= lo) & (d16 < lo + HALF)
            m = (e16 > 0.5) & owned & (eid < E)
            zidx = N + (eid & (ZP - 1))
            srcm = jnp.where(m, s16, zidx)
            dloc = jnp.clip(d16 - lo, 0, HALF - 1)
            r = j // 8
            cs = (j % 8) * 16
            gidx[r, pl.ds(cs, 16)] = srcm
            didx[r, pl.ds(cs, 16)] = dloc
            return _

        lax.fori_loop(0, CHUNK // 16, vec_body, None)

        pltpu.async_copy(g_h.at[gidx.at[0]], rows.at[0], gsem)

        def sc_body(j, _):
            jb = j % 2
            pltpu.make_async_copy(g_h.at[gidx.at[j]], rows.at[jb], gsem).wait()

            @pl.when(j > 0)
            def _():
                pltpu.make_async_copy(rows.at[1 - jb],
                                      acc_s.at[didx.at[j - 1]], ssem).wait()

            @pl.when(j < NSUB - 1)
            def _():
                pltpu.async_copy(g_h.at[gidx.at[j + 1]], rows.at[1 - jb], gsem)

            pltpu.async_copy(rows.at[jb], acc_s.at[didx.at[j]], ssem, add=True)
            return _

        lax.fori_loop(0, NSUB, sc_body, None)
        pltpu.make_async_copy(rows.at[(NSUB - 1) % 2],
                              acc_s.at[didx.at[NSUB - 1]], ssem).wait()
        return _

    lax.fori_loop(0, NCH_B, chunk_body, None)
    plsc.subcore_barrier()

    @pl.when(s == 0)
    def _():
        pltpu.sync_copy(acc_s, acc_out.at[pl.ds(lo, HALF)])


BR = 4000
GRID = N // BR


def _pre_body(h_ref, w_ref, d0_ref, d1_ref, k_ref, g_ref, dinv_ref):
    deg = d0_ref[...] + d1_ref[...] + k_ref[...]
    dinv = jnp.where(deg > 0, lax.rsqrt(deg), 0.0)
    hw = jnp.dot(h_ref[...], w_ref[...], preferred_element_type=jnp.float32)
    g_ref[...] = hw * dinv
    dinv_ref[...] = dinv


def _tc_pre(h, W, d0, d1, keep):
    din = h.shape[1]
    row = lambda i: (i, 0)
    full = lambda i: (0, 0)
    return pl.pallas_call(
        _pre_body,
        grid=(GRID,),
        in_specs=[
            pl.BlockSpec((BR, din), row),
            pl.BlockSpec((din, D), full),
            pl.BlockSpec((BR, 1), row),
            pl.BlockSpec((BR, 1), row),
            pl.BlockSpec((BR, 1), row),
        ],
        out_specs=(pl.BlockSpec((BR, D), row), pl.BlockSpec((BR, 1), row)),
        out_shape=(
            jax.ShapeDtypeStruct((N, D), jnp.float32),
            jax.ShapeDtypeStruct((N, 1), jnp.float32),
        ),
    )(h, W, d0, d1, keep)


def _post_body(acc_ref, g_ref, dinv_ref, k_ref, b_ref, p_ref,
               h_ref, sc_ref):
    dinv = dinv_ref[...]
    pre = (dinv * acc_ref[...] + (dinv * dinv * k_ref[...]) * g_ref[...]
           + b_ref[...])
    hr = jnp.maximum(pre, 0.0)
    sc = jnp.tanh(jnp.dot(hr, p_ref[...], preferred_element_type=jnp.float32))
    h_ref[...] = hr * sc
    sc_ref[...] = sc


def _tc_post(acc, g, dinv, keep, b, p_unit):
    row = lambda i: (i, 0)
    full = lambda i: (0, 0)
    return pl.pallas_call(
        _post_body,
        grid=(GRID,),
        in_specs=[
            pl.BlockSpec((BR, D), row),
            pl.BlockSpec((BR, D), row),
            pl.BlockSpec((BR, 1), row),
            pl.BlockSpec((BR, 1), row),
            pl.BlockSpec((1, D), full),
            pl.BlockSpec((D, 1), full),
        ],
        out_specs=(pl.BlockSpec((BR, D), row), pl.BlockSpec((BR, 1), row)),
        out_shape=(
            jax.ShapeDtypeStruct((N, D), jnp.float32),
            jax.ShapeDtypeStruct((N, 1), jnp.float32),
        ),
    )(acc, g, dinv, keep, b, p_unit)


def _topk_keep(score, batch, node_mask):
    n = score.shape[0]
    bigb = jnp.where(node_mask, batch, NG)
    order = jnp.lexsort((-score, bigb))
    counts_all = jnp.zeros((NG + 1,), jnp.int32).at[bigb].add(1)
    starts = jnp.cumsum(counts_all) - counts_all
    bg = bigb[order]
    ranks = jnp.arange(n) - starts[bg]
    k = jnp.ceil(RATIO * counts_all[:NG].astype(jnp.float32)).astype(jnp.int32)
    k_ext = jnp.concatenate([k, jnp.zeros((1,), jnp.int32)])
    keep_sorted = ranks < k_ext[bg]
    return jnp.zeros((n,), bool).at[order].set(keep_sorted)


def _readout(x, batch, node_mask):
    bb = jnp.where(node_mask, batch, NG)
    mx = jax.ops.segment_max(x, bb, num_segments=NG + 1)[:NG]
    sm = jax.ops.segment_sum(x, bb, num_segments=NG + 1)[:NG]
    cnt = jax.ops.segment_sum(jnp.ones((x.shape[0],), x.dtype), bb,
                              num_segments=NG + 1)[:NG]
    mean = sm / jnp.maximum(cnt, 1.0)[:, None]
    return jnp.concatenate([mx, mean], axis=1)


def kernel(x, edge_index, batch, W1, b1, p1, W2, b2, p2, W3, b3, p3,
           lW1, lb1, lW2, lb2):
    pad = jnp.zeros((E_PAD - E,), jnp.int32)
    src_p = jnp.concatenate([edge_index[0], pad])
    dst_p = jnp.concatenate([edge_index[1], pad])
    em = jnp.ones((E_PAD,), jnp.float32)
    keep_b = jnp.ones((N,), bool)
    zn = jnp.zeros((N,), jnp.float32)
    zh = jnp.zeros((HALF, D), jnp.float32)

    h = x
    outs = []
    for (W, b, p) in ((W1, b1, p1), (W2, b2, p2), (W3, b3, p3)):
        keep_f = keep_b.astype(jnp.float32)
        em, deg_parts = _edge_mask_deg(src_p, dst_p, em, keep_f, zn)
        g, dinv = _tc_pre(h, W, deg_parts[0][:, None], deg_parts[1][:, None],
                          keep_f[:, None])
        g_pad = jnp.concatenate([g, jnp.zeros((ZP, D), jnp.float32)], axis=0)
        acc = _messages(src_p, dst_p, em, g_pad, zh)
        p_unit = (p / jnp.linalg.norm(p))[:, None]
        h, score = _tc_post(acc, g, dinv, keep_f[:, None], b[None, :], p_unit)
        score = score[:, 0]
        keep_b = _topk_keep(score, batch, keep_b)
        outs.append(_readout(h, batch, keep_b))

    z = outs[0] + outs[1] + outs[2]
    z = jax.nn.relu(z @ lW1.T + lb1)
    return jax.nn.sigmoid(z @ lW2.T + lb2)
